# baseline (speedup 1.0000x reference)
import dataclasses

import jax
import jax.numpy as jnp
from jax import lax
from jax.experimental import pallas as pl
from jax.experimental.pallas import tpu as pltpu
from jax.experimental.pallas import tpu_sc as plsc

_CP = pltpu.CompilerParams()
for _f, _v in (("needs_layout_passes", False), ("use_tc_tiling_on_sc", True)):
    if _f in pltpu.CompilerParams.__dataclass_fields__:
        _CP = dataclasses.replace(_CP, **{_f: _v})

D = 8
B, C, P = 16, 4, 2
T = 262144
LANES = 16

B_TC = 8
B_SC = B - B_TC

CH = 16384
N_CHUNKS = T // CH
OUT_CH = CH // D

CHT = 16384
NCHT = T // CHT
SEG = 1024


def _sc_downsample(x):
    mesh = plsc.VectorSubcoreMesh(core_axis_name="core",
                                  subcore_axis_name="subcore")

    @pl.kernel(out_type=jax.ShapeDtypeStruct((B_SC, C, P, T // D),
                                             jnp.float32),
               mesh=mesh, compiler_params=_CP)
    def k(x_hbm, o_hbm):
        def body(in_vmem, out_vmem):
            base = lax.iota(jnp.int32, LANES) * D
            zero = jnp.zeros((LANES,), jnp.int32)

            for p in range(P):
                pvec = jnp.full((LANES,), p, jnp.int32)

                @pl.loop(0, OUT_CH // LANES)
                def _(j, pvec=pvec, p=p):
                    idx = base + j * (D * LANES)
                    vals = plsc.load_gather(in_vmem, [zero, zero, pvec, idx])
                    out_vmem[0, 0, p, pl.ds(j * LANES, LANES)] = vals

        pltpu.emit_pipeline(
            body,
            grid=(B_SC * C * N_CHUNKS,),
            in_specs=[pl.BlockSpec(
                (1, 1, P, CH),
                lambda f: (B_TC + f // (C * N_CHUNKS),
                           (f // N_CHUNKS) % C, 0, f % N_CHUNKS))],
            out_specs=[pl.BlockSpec(
                (1, 1, P, OUT_CH),
                lambda f: (f // (C * N_CHUNKS),
                           (f // N_CHUNKS) % C, 0, f % N_CHUNKS))],
            core_axis_name=("core", "subcore"),
            dimension_semantics=(pltpu.PARALLEL,),
        )(x_hbm, o_hbm)

    return k(x)


def _tc_body(x_ref, o_ref):
    v = x_ref[0, 0]
    sel = (lax.broadcasted_iota(jnp.int32, (SEG, SEG // D), 0)
           == D * lax.broadcasted_iota(jnp.int32, (SEG, SEG // D), 1)
           ).astype(jnp.float32)
    v2 = v.reshape(P * (CHT // SEG), SEG)
    o = jnp.dot(v2, sel, preferred_element_type=jnp.float32)
    o_ref[0, 0] = o.reshape(P, CHT // D)


def _tc_downsample(x):
    return pl.pallas_call(
        _tc_body,
        grid=(B_TC, C, NCHT),
        in_specs=[pl.BlockSpec((1, 1, P, CHT), lambda i, j, k: (i, j, 0, k))],
        out_specs=pl.BlockSpec((1, 1, P, CHT // D),
                               lambda i, j, k: (i, j, 0, k)),
        out_shape=jax.ShapeDtypeStruct((B_TC, C, P, T // D), jnp.float32),
        compiler_params=pltpu.CompilerParams(
            dimension_semantics=("parallel", "parallel", "arbitrary")),
    )(x)


def kernel(x):
    out_sc = _sc_downsample(x)
    out_tc = _tc_downsample(x)
    return jnp.concatenate([out_tc, out_sc], axis=0)


# speedup vs baseline: 4.1306x; 4.1306x over previous
import dataclasses

import jax
import jax.numpy as jnp
from jax import lax
from jax.experimental import pallas as pl
from jax.experimental.pallas import tpu as pltpu
from jax.experimental.pallas import tpu_sc as plsc

_CP = pltpu.CompilerParams()
for _f, _v in (("needs_layout_passes", False), ("use_tc_tiling_on_sc", True)):
    if _f in pltpu.CompilerParams.__dataclass_fields__:
        _CP = dataclasses.replace(_CP, **{_f: _v})

D = 8
B, C, P = 16, 4, 2
T = 262144
CH = 16384
N_CHUNKS = T // CH
OUT_CH = CH // D
LANES = 16


def _sc_downsample(x):
    mesh = plsc.VectorSubcoreMesh(core_axis_name="core",
                                  subcore_axis_name="subcore")

    @pl.kernel(out_type=jax.ShapeDtypeStruct((B, C, P, T // D), jnp.float32),
               mesh=mesh, compiler_params=_CP)
    def k(x_hbm, o_hbm):
        def body(in_vmem, out_vmem):
            base = lax.iota(jnp.int32, LANES) * D
            zero = jnp.zeros((LANES,), jnp.int32)

            for p in range(P):
                pvec = jnp.full((LANES,), p, jnp.int32)

                @plsc.parallel_loop(0, OUT_CH // LANES, unroll=8)
                def _(j, pvec=pvec, p=p):
                    idx = base + j * (D * LANES)
                    vals = plsc.load_gather(in_vmem, [zero, zero, pvec, idx])
                    out_vmem[0, 0, p, pl.ds(j * LANES, LANES)] = vals

        pltpu.emit_pipeline(
            body,
            grid=(B * C, N_CHUNKS),
            in_specs=[pl.BlockSpec((1, 1, P, CH),
                                   lambda f, k: (f // C, f % C, 0, k))],
            out_specs=[pl.BlockSpec((1, 1, P, OUT_CH),
                                    lambda f, k: (f // C, f % C, 0, k))],
            core_axis_name=("core", "subcore"),
            dimension_semantics=(pltpu.PARALLEL, pltpu.PARALLEL),
        )(x_hbm, o_hbm)

    return k(x)


def kernel(x):
    return _sc_downsample(x)


# speedup vs baseline: 4.1338x; 1.0008x over previous
import dataclasses

import jax
import jax.numpy as jnp
from jax import lax
from jax.experimental import pallas as pl
from jax.experimental.pallas import tpu as pltpu
from jax.experimental.pallas import tpu_sc as plsc

_CP = pltpu.CompilerParams()
for _f, _v in (("needs_layout_passes", False), ("use_tc_tiling_on_sc", True)):
    if _f in pltpu.CompilerParams.__dataclass_fields__:
        _CP = dataclasses.replace(_CP, **{_f: _v})

D = 8
B, C, P = 16, 4, 2
T = 262144
CH = 16384
N_CHUNKS = T // CH
OUT_CH = CH // D
LANES = 16


def _sc_downsample(x):
    mesh = plsc.VectorSubcoreMesh(core_axis_name="core",
                                  subcore_axis_name="subcore")

    @pl.kernel(out_type=jax.ShapeDtypeStruct((B, C, P, T // D), jnp.float32),
               mesh=mesh, compiler_params=_CP)
    def k(x_hbm, o_hbm):
        def body(in_vmem, out_vmem):
            base = lax.iota(jnp.int32, LANES) * D
            zero = jnp.zeros((LANES,), jnp.int32)

            for p in range(P):
                pvec = jnp.full((LANES,), p, jnp.int32)

                @plsc.parallel_loop(0, OUT_CH // LANES, unroll=8)
                def _(j, pvec=pvec, p=p):
                    idx = base + j * (D * LANES)
                    vals = plsc.load_gather(in_vmem, [zero, zero, pvec, idx])
                    out_vmem[0, 0, p, pl.ds(j * LANES, LANES)] = vals

        pltpu.emit_pipeline(
            body,
            grid=(B * C, N_CHUNKS),
            in_specs=[pl.BlockSpec((1, 1, P, CH),
                                   lambda f, k: (f // C, f % C, 0, k))],
            out_specs=[pl.BlockSpec((1, 1, P, OUT_CH),
                                    lambda f, k: (f // C, f % C, 0, k))],
            core_axis_name=("core", "subcore"),
            dimension_semantics=(pltpu.PARALLEL, pltpu.PARALLEL),
            trace_scopes=False,
        )(x_hbm, o_hbm)

    return k(x)


def kernel(x):
    return _sc_downsample(x)


# speedup vs baseline: 4.5220x; 1.0939x over previous
import dataclasses

import jax
import jax.numpy as jnp
from jax import lax
from jax.experimental import pallas as pl
from jax.experimental.pallas import tpu as pltpu
from jax.experimental.pallas import tpu_sc as plsc

_CP = pltpu.CompilerParams()
for _f, _v in (("needs_layout_passes", False), ("use_tc_tiling_on_sc", True)):
    if _f in pltpu.CompilerParams.__dataclass_fields__:
        _CP = dataclasses.replace(_CP, **{_f: _v})

D = 8
B, C, P = 16, 4, 2
T = 262144
CH = 8192
N_CHUNKS = T // CH
OUT_CH = CH // D
LANES = 16
NBUF = 4
NW = 32
PAIRS_PER_W = (B * C) // NW
BLOCKS = PAIRS_PER_W * N_CHUNKS


def _sc_downsample(x):
    mesh = plsc.VectorSubcoreMesh(core_axis_name="core",
                                  subcore_axis_name="subcore")

    @pl.kernel(out_type=jax.ShapeDtypeStruct((B, C, P, T // D), jnp.float32),
               mesh=mesh, compiler_params=_CP,
               scratch_types=[
                   pltpu.VMEM((NBUF, P, CH), jnp.float32),
                   pltpu.VMEM((NBUF, P, OUT_CH), jnp.float32),
                   pltpu.SemaphoreType.DMA((NBUF,)),
                   pltpu.SemaphoreType.DMA((NBUF,)),
               ])
    def k(x_hbm, o_hbm, inb, outb, insem, outsem):
        wid = lax.axis_index("subcore") * 2 + lax.axis_index("core")
        f0 = wid * PAIRS_PER_W

        def addr(g):
            f = f0 + g // N_CHUNKS
            kk = g % N_CHUNKS
            return f // C, f % C, kk

        def start_in(g, i):
            bb, cc, kk = addr(g)
            pltpu.async_copy(
                x_hbm.at[bb, cc, :, pl.ds(kk * CH, CH)],
                inb.at[i], insem.at[i])

        def wait_in(g, i):
            bb, cc, kk = addr(g)
            pltpu.make_async_copy(
                x_hbm.at[bb, cc, :, pl.ds(kk * CH, CH)],
                inb.at[i], insem.at[i]).wait()

        def start_out(g, i):
            bb, cc, kk = addr(g)
            pltpu.async_copy(
                outb.at[i],
                o_hbm.at[bb, cc, :, pl.ds(kk * OUT_CH, OUT_CH)],
                outsem.at[i])

        def wait_out(g, i):
            bb, cc, kk = addr(g)
            pltpu.make_async_copy(
                outb.at[i],
                o_hbm.at[bb, cc, :, pl.ds(kk * OUT_CH, OUT_CH)],
                outsem.at[i]).wait()

        base = lax.iota(jnp.int32, LANES) * D

        def compute(i):
            for p in range(P):
                pvec = jnp.full((LANES,), p, jnp.int32)

                @plsc.parallel_loop(0, OUT_CH // LANES, unroll=8)
                def _(j, pvec=pvec, p=p, i=i):
                    idx = base + j * (D * LANES)
                    vals = plsc.load_gather(inb.at[i], [pvec, idx])
                    outb[i, p, pl.ds(j * LANES, LANES)] = vals

        for b in range(NBUF):
            start_in(b, b)

        @pl.loop(0, BLOCKS, step=NBUF)
        def _(g0):
            for b in range(NBUF):
                g = g0 + b

                @pl.when(g0 > 0)
                def _(g=g, b=b):
                    wait_out(g - NBUF, b)

                wait_in(g, b)
                compute(b)
                start_out(g, b)

                @pl.when(g + NBUF < BLOCKS)
                def _(g=g, b=b):
                    start_in(g + NBUF, b)

        for b in range(NBUF):
            wait_out(BLOCKS - NBUF + b, b)

    return k(x)


def kernel(x):
    return _sc_downsample(x)


# speedup vs baseline: 4.6231x; 1.0223x over previous
import dataclasses

import jax
import jax.numpy as jnp
from jax import lax
from jax.experimental import pallas as pl
from jax.experimental.pallas import tpu as pltpu
from jax.experimental.pallas import tpu_sc as plsc

_CP = pltpu.CompilerParams()
for _f, _v in (("needs_layout_passes", False), ("use_tc_tiling_on_sc", True)):
    if _f in pltpu.CompilerParams.__dataclass_fields__:
        _CP = dataclasses.replace(_CP, **{_f: _v})

D = 8
B, C, P = 16, 4, 2
T = 262144
CH = 4096
N_CHUNKS = T // CH
OUT_CH = CH // D
LANES = 16
NBUF = 8
NW = 32
PAIRS_PER_W = (B * C) // NW
BLOCKS = PAIRS_PER_W * N_CHUNKS


def _sc_downsample(x):
    mesh = plsc.VectorSubcoreMesh(core_axis_name="core",
                                  subcore_axis_name="subcore")

    @pl.kernel(out_type=jax.ShapeDtypeStruct((B, C, P, T // D), jnp.float32),
               mesh=mesh, compiler_params=_CP,
               scratch_types=[
                   pltpu.VMEM((NBUF, P, CH), jnp.float32),
                   pltpu.VMEM((NBUF, P, OUT_CH), jnp.float32),
                   pltpu.SemaphoreType.DMA((NBUF,)),
                   pltpu.SemaphoreType.DMA((NBUF,)),
               ])
    def k(x_hbm, o_hbm, inb, outb, insem, outsem):
        wid = lax.axis_index("subcore") * 2 + lax.axis_index("core")
        f0 = wid * PAIRS_PER_W

        def addr(g):
            f = f0 + g // N_CHUNKS
            kk = g % N_CHUNKS
            return f // C, f % C, kk

        def start_in(g, i):
            bb, cc, kk = addr(g)
            pltpu.async_copy(
                x_hbm.at[bb, cc, :, pl.ds(kk * CH, CH)],
                inb.at[i], insem.at[i])

        def wait_in(g, i):
            bb, cc, kk = addr(g)
            pltpu.make_async_copy(
                x_hbm.at[bb, cc, :, pl.ds(kk * CH, CH)],
                inb.at[i], insem.at[i]).wait()

        def start_out(g, i):
            bb, cc, kk = addr(g)
            pltpu.async_copy(
                outb.at[i],
                o_hbm.at[bb, cc, :, pl.ds(kk * OUT_CH, OUT_CH)],
                outsem.at[i])

        def wait_out(g, i):
            bb, cc, kk = addr(g)
            pltpu.make_async_copy(
                outb.at[i],
                o_hbm.at[bb, cc, :, pl.ds(kk * OUT_CH, OUT_CH)],
                outsem.at[i]).wait()

        base = lax.iota(jnp.int32, LANES) * D

        def compute(i):
            for p in range(P):
                pvec = jnp.full((LANES,), p, jnp.int32)

                @plsc.parallel_loop(0, OUT_CH // LANES, unroll=8)
                def _(j, pvec=pvec, p=p, i=i):
                    idx = base + j * (D * LANES)
                    vals = plsc.load_gather(inb.at[i], [pvec, idx])
                    outb[i, p, pl.ds(j * LANES, LANES)] = vals

        for b in range(NBUF):
            start_in(b, b)

        @pl.loop(0, BLOCKS, step=NBUF)
        def _(g0):
            for b in range(NBUF):
                g = g0 + b

                @pl.when(g0 > 0)
                def _(g=g, b=b):
                    wait_out(g - NBUF, b)

                wait_in(g, b)
                compute(b)
                start_out(g, b)

                @pl.when(g + NBUF < BLOCKS)
                def _(g=g, b=b):
                    start_in(g + NBUF, b)

        for b in range(NBUF):
            wait_out(BLOCKS - NBUF + b, b)

    return k(x)


def kernel(x):
    return _sc_downsample(x)
